# baseline (speedup 1.0000x reference)
import functools

import jax
import jax.numpy as jnp
from jax import lax
from jax.experimental import pallas as pl
from jax.experimental.pallas import tpu as pltpu
from jax.experimental.pallas import tpu_sc as plsc

VOCAB_SIZE = 1000000
EMBEDDING_DIM = 16
NUM_NS = 4
BATCH = 16384

_NC = 2
_NS = 16
_NW = _NC * _NS
_LANES = 16
_D = EMBEDDING_DIM
_WIDE = 128
_PACK = _WIDE // _D

_B_PER_W = BATCH // _NW
_J_PER_W = _B_PER_W * (NUM_NS + 1)
_ICHUNK = 128
_CCHUNK = 256
_NCHUNKS = _J_PER_W // _CCHUNK


def _sc_kernel(tgt_idx_hbm, ctx_idx_hbm, ttab_hbm, ctab_hbm, out_hbm,
               tgt_idx_v, ctx_idx_v, tgt_widx_v, ctx_widx_v,
               wide0_v, wide1_v, tgt_c_v, out_v, sem):
    wid = lax.axis_index("s") * _NC + lax.axis_index("c")
    b_base = wid * _B_PER_W
    j_base = wid * _J_PER_W

    pltpu.sync_copy(tgt_idx_hbm.at[pl.ds(b_base, _B_PER_W)], tgt_idx_v)
    pltpu.sync_copy(ctx_idx_hbm.at[pl.ds(j_base, _J_PER_W)], ctx_idx_v)

    lanes = lax.iota(jnp.int32, _LANES)

    def shift_pass(i, _):
        tgt_widx_v[pl.ds(i * _LANES, _LANES)] = lax.shift_right_logical(
            tgt_idx_v[pl.ds(i * _LANES, _LANES)], 3)
        return 0

    def shift_pass_ctx(i, _):
        ctx_widx_v[pl.ds(i * _LANES, _LANES)] = lax.shift_right_logical(
            ctx_idx_v[pl.ds(i * _LANES, _LANES)], 3)
        return 0

    lax.fori_loop(0, _B_PER_W // _LANES, shift_pass, 0)
    lax.fori_loop(0, _J_PER_W // _LANES, shift_pass_ctx, 0)

    tcopies = []
    for k in range(_B_PER_W // _ICHUNK):
        buf = wide0_v if k < 2 else wide1_v
        row = (k % 2) * _ICHUNK
        tcopies.append(pltpu.async_copy(
            ttab_hbm.at[tgt_widx_v.at[pl.ds(k * _ICHUNK, _ICHUNK)]],
            buf.at[pl.ds(row, _ICHUNK)], sem))
    for c in tcopies:
        c.wait()

    def make_compact(buf, half):
        def compact(k, _):
            row = lanes + k * _LANES
            gb = half * 256 + k * _LANES + lanes
            v = tgt_idx_v[pl.ds(half * 256 + k * _LANES, _LANES)]
            s = lax.mul(lax.bitwise_and(v, _PACK - 1), _D)
            prow = lax.shift_right_logical(gb, 3)
            pcol = lax.mul(lax.bitwise_and(gb, _PACK - 1), _D)
            for d in range(_D):
                val = plsc.load_gather(buf, [row, s + d])
                plsc.store_scatter(tgt_c_v, [prow, pcol + d], val)
            return 0
        return compact

    lax.fori_loop(0, 16, make_compact(wide0_v, 0), 0)
    lax.fori_loop(0, 16, make_compact(wide1_v, 1), 0)

    def fire(c, buf):
        h = []
        for k in range(_CCHUNK // _ICHUNK):
            h.append(pltpu.async_copy(
                ctab_hbm.at[ctx_widx_v.at[
                    pl.ds(c * _CCHUNK + k * _ICHUNK, _ICHUNK)]],
                buf.at[pl.ds(k * _ICHUNK, _ICHUNK)], sem))
        return h

    def make_compute(c, buf):
        def block(k, _):
            jloc = k * _LANES + lanes
            j = c * _CCHUNK + k * _LANES + lanes
            v = ctx_idx_v[pl.ds(c * _CCHUNK + k * _LANES, _LANES)]
            s = lax.mul(lax.bitwise_and(v, _PACK - 1), _D)
            b = lax.div(j, NUM_NS + 1)
            prow = lax.shift_right_logical(b, 3)
            pcol = lax.mul(lax.bitwise_and(b, _PACK - 1), _D)
            acc = jnp.zeros((_LANES,), jnp.float32)
            for d in range(_D):
                cv = plsc.load_gather(buf, [jloc, s + d])
                tv = plsc.load_gather(tgt_c_v, [prow, pcol + d])
                acc = acc + cv * tv
            out_v[pl.ds(c * _CCHUNK + k * _LANES, _LANES)] = acc
            return 0
        return block

    bufs = (wide0_v, wide1_v)
    pending = fire(0, bufs[0])
    for c in range(_NCHUNKS):
        nxt = fire(c + 1, bufs[(c + 1) % 2]) if c + 1 < _NCHUNKS else []
        for h in pending:
            h.wait()
        lax.fori_loop(0, _CCHUNK // _LANES, make_compute(c, bufs[c % 2]), 0)
        pending = nxt

    pltpu.sync_copy(out_v, out_hbm.at[pl.ds(j_base, _J_PER_W)])


@jax.jit
def kernel(target, context, target_table, context_table):
    tgt_idx = target.reshape(BATCH)
    ctx_idx = context.reshape(BATCH * (NUM_NS + 1))
    wide_shape = (VOCAB_SIZE * _D // _WIDE, _WIDE)

    run = pl.kernel(
        _sc_kernel,
        out_type=jax.ShapeDtypeStruct((BATCH * (NUM_NS + 1),), jnp.float32),
        mesh=plsc.VectorSubcoreMesh(core_axis_name="c", subcore_axis_name="s"),
        compiler_params=pltpu.CompilerParams(
            needs_layout_passes=False, use_tc_tiling_on_sc=True),
        scratch_types=[
            pltpu.VMEM((_B_PER_W,), jnp.int32),
            pltpu.VMEM((_J_PER_W,), jnp.int32),
            pltpu.VMEM((_B_PER_W,), jnp.int32),
            pltpu.VMEM((_J_PER_W,), jnp.int32),
            pltpu.VMEM((_CCHUNK, _WIDE), jnp.float32),
            pltpu.VMEM((_CCHUNK, _WIDE), jnp.float32),
            pltpu.VMEM((_B_PER_W // _PACK, _WIDE), jnp.float32),
            pltpu.VMEM((_J_PER_W,), jnp.float32),
            pltpu.SemaphoreType.DMA,
        ],
    )
    out = run(tgt_idx, ctx_idx,
              target_table.reshape(wide_shape),
              context_table.reshape(wide_shape))
    return out.reshape(BATCH, NUM_NS + 1)
